# baseline (speedup 1.0000x reference)
import functools

import jax
import jax.numpy as jnp
from jax import lax
from jax.experimental import pallas as pl
from jax.experimental.pallas import tpu as pltpu
from jax.experimental.pallas import tpu_sc as plsc

B = 8
HB = 4
C = 160
LAST = 256
N = 512
K = 16
NBLK = 4
CH1 = 256
NW = 32
ROWS_C = 128
HBAGS = HB * N
HCHUNKS = HBAGS * K // (NW * ROWS_C)

_F32 = jnp.float32
_BF16 = jnp.bfloat16
_INF = float('inf')


def _dot(a, b, dims):
    return lax.dot_general(a.astype(_BF16), b.astype(_BF16), (dims, ((), ())),
                           preferred_element_type=_F32)


def _knn_select(X, boff):
    xs_cm = X.T
    nrm = jnp.sqrt(jnp.sum(xs_cm * xs_cm, axis=0, keepdims=True))
    v = xs_cm / (nrm + 1e-12)
    sq = jnp.sum(v * v, axis=0)
    vt = v.T
    G = _dot(vt, vt, ((1,), (1,)))
    d = (sq[:, None] + sq[None, :]) - 2.0 * G

    iota = lax.broadcasted_iota(jnp.int32, (N, N), 1)
    rows = []
    Mb = jnp.zeros((N, N), dtype=jnp.bool_)
    for _ in range(K):
        amin = jnp.argmin(jnp.where(Mb, _INF, d), axis=1).astype(jnp.int32)
        rows.append(amin + boff)
        Mb = Mb | (iota == amin[:, None])
    return jnp.stack(rows)


def _edge_ffn(X, xj_ref, Wg_ref, bg_ref, sg_ref, bb_ref,
              W1_ref, b1_ref, s1_ref, be1_ref, W2_ref, b2_ref, s2_ref, be2_ref, i):
    CH = 2048
    NNC = CH // K
    Wg = Wg_ref[i].astype(_BF16)
    bg, sg, bb = bg_ref[i], sg_ref[i], bb_ref[i]
    ychunks = []
    for c0 in range(0, N * K, CH):
        Xj = xj_ref[0, pl.ds(c0, CH), :]
        n0 = c0 // K
        Xi = jnp.broadcast_to(X[n0:n0 + NNC][:, None, :], (NNC, K, C)).reshape(CH, C)
        feat = jnp.concatenate([Xi, Xj - Xi], axis=1)
        yk = lax.dot_general(feat.astype(_BF16), Wg, (((1,), (0,)), ((), ())),
                             preferred_element_type=_F32)
        yk = jnp.maximum((yk + bg) * sg + bb, 0.0)
        ychunks.append(jnp.max(yk.reshape(NNC, K, C), axis=1))
    Y = jnp.concatenate(ychunks, axis=0)
    Xe = X + Y
    H = (_dot(Xe, W1_ref[i], ((1,), (0,))) + b1_ref[i]) * s1_ref[i] + be1_ref[i]
    H = jnp.maximum(H, 0.0)
    return (_dot(H, W2_ref[i], ((1,), (0,))) + b2_ref[i]) * s2_ref[i] + be2_ref[i]


def _tc_front_body(xrT_ref, WcdT_ref, bcd_ref, X_ref, idx_ref):
    X = _dot(WcdT_ref[...], xrT_ref[0], ((1,), (0,))) + bcd_ref[...]
    X_ref[0] = X
    idx_ref[0] = _knn_select(X, pl.program_id(0) * N)


def _tc_block_body(i, last, base_b, X_in_ref, xj_ref, Wg_ref, bg_ref, sg_ref,
                   bb_ref, W1_ref, b1_ref, s1_ref, be1_ref, W2_ref, b2_ref,
                   s2_ref, be2_ref, Wc1T_ref, bc1_ref, Wc2T_ref, bc2_ref,
                   *out_refs):
    X = X_in_ref[0]
    Xn = _edge_ffn(X, xj_ref, Wg_ref, bg_ref, sg_ref, bb_ref, W1_ref, b1_ref,
                   s1_ref, be1_ref, W2_ref, b2_ref, s2_ref, be2_ref, i)
    if last:
        h1 = jnp.maximum(_dot(Wc1T_ref[...], Xn, ((1,), (0,))) + bc1_ref[...], 0.0)
        o = _dot(Wc2T_ref[...], h1, ((1,), (0,))) + bc2_ref[...]
        out_refs[0][0] = jnp.maximum(o, 0.0)
    else:
        out_refs[0][0] = Xn
        out_refs[1][0] = _knn_select(Xn, (base_b + pl.program_id(0)) * N)


def _sc_gather_body(x_hbm, idx_hbm, out_hbm, idx_v, buf0, buf1, sem0, sem1):
    wid = lax.axis_index("s") * 2 + lax.axis_index("c")
    pltpu.sync_copy(idx_hbm.at[wid], idx_v)
    bufs = (buf0, buf1)
    sems = (sem0, sem1)
    obase = wid * HCHUNKS * ROWS_C
    pend = [None, None]
    pend[0] = pltpu.async_copy(x_hbm.at[idx_v.at[0]], bufs[0], sems[0])
    for c in range(HCHUNKS):
        s = c % 2
        if c + 1 < HCHUNKS:
            pend[1 - s] = pltpu.async_copy(x_hbm.at[idx_v.at[c + 1]],
                                           bufs[1 - s], sems[1 - s])
        pend[s].wait()
        pltpu.sync_copy(bufs[s], out_hbm.at[pl.ds(obase + c * ROWS_C, ROWS_C)])


_sc_gather = pl.kernel(
    _sc_gather_body,
    out_type=jax.ShapeDtypeStruct((HBAGS * K, C), _F32),
    mesh=plsc.VectorSubcoreMesh(core_axis_name="c", subcore_axis_name="s",
                                num_cores=2, num_subcores=16),
    scratch_types=[pltpu.VMEM((HCHUNKS, ROWS_C), jnp.int32),
                   pltpu.VMEM((ROWS_C, C), _F32),
                   pltpu.VMEM((ROWS_C, C), _F32),
                   pltpu.SemaphoreType.DMA,
                   pltpu.SemaphoreType.DMA],
    compiler_params=pltpu.CompilerParams(use_tc_tiling_on_sc=False),
)


def _reorder_idx(idx_half):
    return jnp.transpose(idx_half, (0, 2, 1)).reshape(NW, HCHUNKS, ROWS_C)


def _stack_params(params):
    inv = 1.0 / jnp.sqrt(1.0 + 1e-5)
    get = lambda n: jnp.stack([params['b%d_%s' % (i, n)] for i in range(NBLK)])
    row = lambda a: a[:, None, :]
    return (get('Wg'), row(get('bg')), row(get('gg') * inv), row(get('bb')),
            get('W1'), row(get('b1')), row(get('g1') * inv), row(get('be1')),
            get('W2'), row(get('b2')), row(get('g2') * inv), row(get('be2')))


def kernel(x, params):
    xrT = x.reshape(B, LAST, C)
    wp = _stack_params(params)
    head = (params['Wc1'].T, params['bc1'][:, None],
            params['Wc2'].T, params['bc2'][:, None])

    bspec = lambda shp: pl.BlockSpec(shp, lambda b: (0,) * len(shp))
    batch3 = lambda s2, s3: pl.BlockSpec((1, s2, s3), lambda b: (b, 0, 0))

    Xall, idx = pl.pallas_call(
        _tc_front_body,
        grid=(B,),
        in_specs=[batch3(LAST, C), bspec((N, LAST)), bspec((N, 1))],
        out_specs=[batch3(N, C), batch3(K, N)],
        out_shape=[jax.ShapeDtypeStruct((B, N, C), _F32),
                   jax.ShapeDtypeStruct((B, K, N), jnp.int32)],
    )(xrT, params['W_cd'].T, params['b_cd'][:, None])

    wspecs = [bspec(a.shape) for a in wp] + [bspec(a.shape) for a in head]
    Xh = [Xall[:HB], Xall[HB:]]
    idxh = [idx[:HB], idx[HB:]]

    for i in range(NBLK):
        last = i == NBLK - 1
        table = jnp.concatenate(Xh).reshape(B * N, C)
        if last:
            out_specs = [pl.BlockSpec((1, 1, C), lambda b: (b, 0, 0))]
            out_shape = [jax.ShapeDtypeStruct((HB, 1, C), _F32)]
        else:
            out_specs = [batch3(N, C), batch3(K, N)]
            out_shape = [jax.ShapeDtypeStruct((HB, N, C), _F32),
                         jax.ShapeDtypeStruct((HB, K, N), jnp.int32)]
        res = []
        for h in range(2):
            xj = _sc_gather(table, _reorder_idx(idxh[h]))
            xj = xj.reshape(HB, N * K, C)
            outs = pl.pallas_call(
                functools.partial(_tc_block_body, i, last, h * HB),
                grid=(HB,),
                in_specs=[batch3(N, C), batch3(N * K, C)] + wspecs,
                out_specs=out_specs,
                out_shape=out_shape,
            )(Xh[h], xj, *wp, *head)
            res.append(outs)
        if last:
            return jnp.concatenate([res[0][0], res[1][0]]).reshape(B, C)
        Xh = [res[0][0], res[1][0]]
        idxh = [res[0][1], res[1][1]]


# speedup vs baseline: 1.0471x; 1.0471x over previous
import functools

import jax
import jax.numpy as jnp
from jax import lax
from jax.experimental import pallas as pl
from jax.experimental.pallas import tpu as pltpu
from jax.experimental.pallas import tpu_sc as plsc

B = 8
HB = 4
C = 160
LAST = 256
N = 512
K = 16
NBLK = 4
CH1 = 256
NW = 32
ROWS_C = 128
BAGS = B * N
CHUNKS = BAGS * K // (NW * ROWS_C)
NBUF = 4

_F32 = jnp.float32
_BF16 = jnp.bfloat16
_INF = float('inf')


def _dot(a, b, dims):
    return lax.dot_general(a.astype(_BF16), b.astype(_BF16), (dims, ((), ())),
                           preferred_element_type=_F32)


def _knn_select(X, boff):
    xs_cm = X.T
    nrm = jnp.sqrt(jnp.sum(xs_cm * xs_cm, axis=0, keepdims=True))
    v = xs_cm / (nrm + 1e-12)
    sq = jnp.sum(v * v, axis=0)
    vt = v.T
    G = _dot(vt, vt, ((1,), (1,)))
    d = (sq[:, None] + sq[None, :]) - 2.0 * G

    iota = lax.broadcasted_iota(jnp.int32, (N, N), 1)
    rows = []
    Mb = jnp.zeros((N, N), dtype=jnp.bool_)
    for _ in range(K):
        amin = jnp.argmin(jnp.where(Mb, _INF, d), axis=1).astype(jnp.int32)
        rows.append(amin + boff)
        Mb = Mb | (iota == amin[:, None])
    return jnp.stack(rows)


def _edge_ffn(X, xj_ref, Wg_ref, bg_ref, sg_ref, bb_ref,
              W1_ref, b1_ref, s1_ref, be1_ref, W2_ref, b2_ref, s2_ref, be2_ref, i):
    CH = 2048
    NNC = CH // K
    Wg = Wg_ref[i].astype(_BF16)
    bg, sg, bb = bg_ref[i], sg_ref[i], bb_ref[i]
    ychunks = []
    for c0 in range(0, N * K, CH):
        Xj = xj_ref[0, pl.ds(c0, CH), :]
        n0 = c0 // K
        Xi = jnp.broadcast_to(X[n0:n0 + NNC][:, None, :], (NNC, K, C)).reshape(CH, C)
        feat = jnp.concatenate([Xi, Xj - Xi], axis=1)
        yk = lax.dot_general(feat.astype(_BF16), Wg, (((1,), (0,)), ((), ())),
                             preferred_element_type=_F32)
        yk = jnp.maximum((yk + bg) * sg + bb, 0.0)
        ychunks.append(jnp.max(yk.reshape(NNC, K, C), axis=1))
    Y = jnp.concatenate(ychunks, axis=0)
    Xe = X + Y
    H = (_dot(Xe, W1_ref[i], ((1,), (0,))) + b1_ref[i]) * s1_ref[i] + be1_ref[i]
    H = jnp.maximum(H, 0.0)
    return (_dot(H, W2_ref[i], ((1,), (0,))) + b2_ref[i]) * s2_ref[i] + be2_ref[i]


def _tc_front_body(xrT_ref, WcdT_ref, bcd_ref, X_ref, idx_ref):
    X = _dot(WcdT_ref[...], xrT_ref[0], ((1,), (0,))) + bcd_ref[...]
    X_ref[0] = X
    idx_ref[0] = _knn_select(X, pl.program_id(0) * N)


def _tc_block_body(i, last, base_b, X_in_ref, xj_ref, Wg_ref, bg_ref, sg_ref,
                   bb_ref, W1_ref, b1_ref, s1_ref, be1_ref, W2_ref, b2_ref,
                   s2_ref, be2_ref, Wc1T_ref, bc1_ref, Wc2T_ref, bc2_ref,
                   *out_refs):
    X = X_in_ref[0]
    Xn = _edge_ffn(X, xj_ref, Wg_ref, bg_ref, sg_ref, bb_ref, W1_ref, b1_ref,
                   s1_ref, be1_ref, W2_ref, b2_ref, s2_ref, be2_ref, i)
    if last:
        h1 = jnp.maximum(_dot(Wc1T_ref[...], Xn, ((1,), (0,))) + bc1_ref[...], 0.0)
        o = _dot(Wc2T_ref[...], h1, ((1,), (0,))) + bc2_ref[...]
        out_refs[0][0] = jnp.maximum(o, 0.0)
    else:
        out_refs[0][0] = Xn
        out_refs[1][0] = _knn_select(Xn, (base_b + pl.program_id(0)) * N)


def _sc_gather_body(x_hbm, idx_hbm, out_hbm, idx_v, *bufsem):
    wid = lax.axis_index("s") * 2 + lax.axis_index("c")
    pltpu.sync_copy(idx_hbm.at[wid], idx_v)
    bufs = bufsem[:NBUF]
    sems = bufsem[NBUF:]
    obase = wid * CHUNKS * ROWS_C
    pend = [None] * NBUF
    for c in range(min(NBUF, CHUNKS)):
        pend[c] = pltpu.async_copy(x_hbm.at[idx_v.at[c]], bufs[c], sems[c])
    for c in range(CHUNKS):
        s = c % NBUF
        pend[s].wait()
        pltpu.sync_copy(bufs[s], out_hbm.at[pl.ds(obase + c * ROWS_C, ROWS_C)])
        n = c + NBUF
        if n < CHUNKS:
            pend[s] = pltpu.async_copy(x_hbm.at[idx_v.at[n]], bufs[s], sems[s])


_sc_gather = pl.kernel(
    _sc_gather_body,
    out_type=jax.ShapeDtypeStruct((BAGS * K, C), _F32),
    mesh=plsc.VectorSubcoreMesh(core_axis_name="c", subcore_axis_name="s",
                                num_cores=2, num_subcores=16),
    scratch_types=[pltpu.VMEM((CHUNKS, ROWS_C), jnp.int32)]
                  + [pltpu.VMEM((ROWS_C, C), _F32)] * NBUF
                  + [pltpu.SemaphoreType.DMA] * NBUF,
    compiler_params=pltpu.CompilerParams(use_tc_tiling_on_sc=False),
)


def _reorder_idx(idx):
    return jnp.transpose(idx, (0, 2, 1)).reshape(NW, CHUNKS, ROWS_C)


def _stack_params(params):
    inv = 1.0 / jnp.sqrt(1.0 + 1e-5)
    get = lambda n: jnp.stack([params['b%d_%s' % (i, n)] for i in range(NBLK)])
    row = lambda a: a[:, None, :]
    return (get('Wg'), row(get('bg')), row(get('gg') * inv), row(get('bb')),
            get('W1'), row(get('b1')), row(get('g1') * inv), row(get('be1')),
            get('W2'), row(get('b2')), row(get('g2') * inv), row(get('be2')))


def kernel(x, params):
    xrT = x.reshape(B, LAST, C)
    wp = _stack_params(params)
    head = (params['Wc1'].T, params['bc1'][:, None],
            params['Wc2'].T, params['bc2'][:, None])

    bspec = lambda shp: pl.BlockSpec(shp, lambda b: (0,) * len(shp))
    batch3 = lambda s2, s3: pl.BlockSpec((1, s2, s3), lambda b: (b, 0, 0))

    Xall, idx = pl.pallas_call(
        _tc_front_body,
        grid=(B,),
        in_specs=[batch3(LAST, C), bspec((N, LAST)), bspec((N, 1))],
        out_specs=[batch3(N, C), batch3(K, N)],
        out_shape=[jax.ShapeDtypeStruct((B, N, C), _F32),
                   jax.ShapeDtypeStruct((B, K, N), jnp.int32)],
    )(xrT, params['W_cd'].T, params['b_cd'][:, None])

    wspecs = [bspec(a.shape) for a in wp] + [bspec(a.shape) for a in head]

    for i in range(NBLK):
        last = i == NBLK - 1
        xj = _sc_gather(Xall.reshape(B * N, C), _reorder_idx(idx))
        xj = xj.reshape(B, N * K, C)
        if last:
            out_specs = [pl.BlockSpec((1, 1, C), lambda b: (b, 0, 0))]
            out_shape = [jax.ShapeDtypeStruct((B, 1, C), _F32)]
        else:
            out_specs = [batch3(N, C), batch3(K, N)]
            out_shape = [jax.ShapeDtypeStruct((B, N, C), _F32),
                         jax.ShapeDtypeStruct((B, K, N), jnp.int32)]
        outs = pl.pallas_call(
            functools.partial(_tc_block_body, i, last, 0),
            grid=(B,),
            in_specs=[batch3(N, C), batch3(N * K, C)] + wspecs,
            out_specs=out_specs,
            out_shape=out_shape,
        )(Xall, xj, *wp, *head)
        if last:
            return outs[0].reshape(B, C)
        Xall, idx = outs


# speedup vs baseline: 1.0540x; 1.0065x over previous
import functools

import jax
import jax.numpy as jnp
from jax import lax
from jax.experimental import pallas as pl
from jax.experimental.pallas import tpu as pltpu
from jax.experimental.pallas import tpu_sc as plsc

B = 8
HB = 4
C = 160
LAST = 256
N = 512
K = 16
NBLK = 4
CH1 = 256
NW = 32
ROWS_C = 128
BAGS = B * N
CHUNKS = BAGS * K // (NW * ROWS_C)
NBUF = 4

_F32 = jnp.float32
_BF16 = jnp.bfloat16
_INF = float('inf')


def _dot(a, b, dims):
    return lax.dot_general(a.astype(_BF16), b.astype(_BF16), (dims, ((), ())),
                           preferred_element_type=_F32)


def _knn_select(X, boff):
    xs_cm = X.T
    nrm = jnp.sqrt(jnp.sum(xs_cm * xs_cm, axis=0, keepdims=True))
    v = xs_cm / (nrm + 1e-12)
    sq = jnp.sum(v * v, axis=0)
    vt = v.T
    G = _dot(vt, vt, ((1,), (1,)))
    d = (sq[:, None] + sq[None, :]) - 2.0 * G

    iota = lax.broadcasted_iota(jnp.int32, (N, N), 1)
    rows = []
    Mb = jnp.zeros((N, N), dtype=jnp.bool_)
    for _ in range(K):
        amin = jnp.argmin(jnp.where(Mb, _INF, d), axis=1).astype(jnp.int32)
        rows.append(amin + boff)
        Mb = Mb | (iota == amin[:, None])
    return jnp.stack(rows)


def _edge_ffn(X, xj_ref, Wg_ref, bg_ref, sg_ref, bb_ref,
              W1_ref, b1_ref, s1_ref, be1_ref, W2_ref, b2_ref, s2_ref, be2_ref, i):
    CH = 2048
    NNC = CH // K
    Wg = Wg_ref[i].astype(_BF16)
    bg, sg, bb = bg_ref[i], sg_ref[i], bb_ref[i]
    ychunks = []
    for c0 in range(0, N * K, CH):
        Xj = xj_ref[0, pl.ds(c0, CH), :]
        n0 = c0 // K
        Xi = jnp.broadcast_to(X[n0:n0 + NNC][:, None, :], (NNC, K, C)).reshape(CH, C)
        Xib = jnp.broadcast_to(X[n0:n0 + NNC].astype(_BF16)[:, None, :],
                               (NNC, K, C)).reshape(CH, C)
        feat = jnp.concatenate([Xib, (Xj - Xi).astype(_BF16)], axis=1)
        yk = lax.dot_general(feat, Wg, (((1,), (0,)), ((), ())),
                             preferred_element_type=_F32)
        yk = jnp.maximum((yk + bg) * sg + bb, 0.0)
        ychunks.append(jnp.max(yk.reshape(NNC, K, C), axis=1))
    Y = jnp.concatenate(ychunks, axis=0)
    Xe = X + Y
    H = (_dot(Xe, W1_ref[i], ((1,), (0,))) + b1_ref[i]) * s1_ref[i] + be1_ref[i]
    H = jnp.maximum(H, 0.0)
    return (_dot(H, W2_ref[i], ((1,), (0,))) + b2_ref[i]) * s2_ref[i] + be2_ref[i]


def _tc_front_body(xrT_ref, WcdT_ref, bcd_ref, X_ref, idx_ref):
    X = _dot(WcdT_ref[...], xrT_ref[0], ((1,), (0,))) + bcd_ref[...]
    X_ref[0] = X
    idx_ref[0] = _knn_select(X, pl.program_id(0) * N)


def _tc_block_body(i, last, base_b, X_in_ref, xj_ref, Wg_ref, bg_ref, sg_ref,
                   bb_ref, W1_ref, b1_ref, s1_ref, be1_ref, W2_ref, b2_ref,
                   s2_ref, be2_ref, Wc1T_ref, bc1_ref, Wc2T_ref, bc2_ref,
                   *out_refs):
    X = X_in_ref[0]
    Xn = _edge_ffn(X, xj_ref, Wg_ref, bg_ref, sg_ref, bb_ref, W1_ref, b1_ref,
                   s1_ref, be1_ref, W2_ref, b2_ref, s2_ref, be2_ref, i)
    if last:
        h1 = jnp.maximum(_dot(Wc1T_ref[...], Xn, ((1,), (0,))) + bc1_ref[...], 0.0)
        o = _dot(Wc2T_ref[...], h1, ((1,), (0,))) + bc2_ref[...]
        out_refs[0][0] = jnp.maximum(o, 0.0)
    else:
        out_refs[0][0] = Xn
        out_refs[1][0] = _knn_select(Xn, (base_b + pl.program_id(0)) * N)


def _sc_gather_body(x_hbm, idx_hbm, out_hbm, idx_v, *bufsem):
    wid = lax.axis_index("s") * 2 + lax.axis_index("c")
    pltpu.sync_copy(idx_hbm.at[wid], idx_v)
    bufs = bufsem[:NBUF]
    sems = bufsem[NBUF:]
    obase = wid * CHUNKS * ROWS_C
    pend = [None] * NBUF
    for c in range(min(NBUF, CHUNKS)):
        pend[c] = pltpu.async_copy(x_hbm.at[idx_v.at[c]], bufs[c], sems[c])
    for c in range(CHUNKS):
        s = c % NBUF
        pend[s].wait()
        pltpu.sync_copy(bufs[s], out_hbm.at[pl.ds(obase + c * ROWS_C, ROWS_C)])
        n = c + NBUF
        if n < CHUNKS:
            pend[s] = pltpu.async_copy(x_hbm.at[idx_v.at[n]], bufs[s], sems[s])


_sc_gather = pl.kernel(
    _sc_gather_body,
    out_type=jax.ShapeDtypeStruct((BAGS * K, C), _F32),
    mesh=plsc.VectorSubcoreMesh(core_axis_name="c", subcore_axis_name="s",
                                num_cores=2, num_subcores=16),
    scratch_types=[pltpu.VMEM((CHUNKS, ROWS_C), jnp.int32)]
                  + [pltpu.VMEM((ROWS_C, C), _F32)] * NBUF
                  + [pltpu.SemaphoreType.DMA] * NBUF,
    compiler_params=pltpu.CompilerParams(use_tc_tiling_on_sc=False),
)


def _reorder_idx(idx):
    return jnp.transpose(idx, (0, 2, 1)).reshape(NW, CHUNKS, ROWS_C)


def _stack_params(params):
    inv = 1.0 / jnp.sqrt(1.0 + 1e-5)
    get = lambda n: jnp.stack([params['b%d_%s' % (i, n)] for i in range(NBLK)])
    row = lambda a: a[:, None, :]
    return (get('Wg'), row(get('bg')), row(get('gg') * inv), row(get('bb')),
            get('W1'), row(get('b1')), row(get('g1') * inv), row(get('be1')),
            get('W2'), row(get('b2')), row(get('g2') * inv), row(get('be2')))


def kernel(x, params):
    xrT = x.reshape(B, LAST, C)
    wp = _stack_params(params)
    head = (params['Wc1'].T, params['bc1'][:, None],
            params['Wc2'].T, params['bc2'][:, None])

    bspec = lambda shp: pl.BlockSpec(shp, lambda b: (0,) * len(shp))
    batch3 = lambda s2, s3: pl.BlockSpec((1, s2, s3), lambda b: (b, 0, 0))

    Xall, idx = pl.pallas_call(
        _tc_front_body,
        grid=(B,),
        in_specs=[batch3(LAST, C), bspec((N, LAST)), bspec((N, 1))],
        out_specs=[batch3(N, C), batch3(K, N)],
        out_shape=[jax.ShapeDtypeStruct((B, N, C), _F32),
                   jax.ShapeDtypeStruct((B, K, N), jnp.int32)],
    )(xrT, params['W_cd'].T, params['b_cd'][:, None])

    wspecs = [bspec(a.shape) for a in wp] + [bspec(a.shape) for a in head]

    for i in range(NBLK):
        last = i == NBLK - 1
        xj = _sc_gather(Xall.reshape(B * N, C), _reorder_idx(idx))
        xj = xj.reshape(B, N * K, C)
        if last:
            out_specs = [pl.BlockSpec((1, 1, C), lambda b: (b, 0, 0))]
            out_shape = [jax.ShapeDtypeStruct((B, 1, C), _F32)]
        else:
            out_specs = [batch3(N, C), batch3(K, N)]
            out_shape = [jax.ShapeDtypeStruct((B, N, C), _F32),
                         jax.ShapeDtypeStruct((B, K, N), jnp.int32)]
        outs = pl.pallas_call(
            functools.partial(_tc_block_body, i, last, 0),
            grid=(B,),
            in_specs=[batch3(N, C), batch3(N * K, C)] + wspecs,
            out_specs=out_specs,
            out_shape=out_shape,
        )(Xall, xj, *wp, *head)
        if last:
            return outs[0].reshape(B, C)
        Xall, idx = outs
